# baseline (speedup 1.0000x reference)
import functools

import jax
import jax.numpy as jnp
import numpy as np
from jax.experimental import pallas as pl

_EPS = 1e-7
_PI = float(np.pi)


def _atan_pos(x):
    t = jnp.abs(x)
    sel1 = t > 2.414213562373095
    sel2 = t > 0.4142135623730950
    xp = jnp.where(sel1, -1.0 / (t + 1e-30), jnp.where(sel2, (t - 1.0) / (t + 1.0), t))
    y0 = jnp.where(sel1, _PI / 2.0, jnp.where(sel2, _PI / 4.0, 0.0))
    z = xp * xp
    r = ((8.05374449538e-2 * z - 1.38776856032e-1) * z + 1.99777106478e-1) * z
    r = (r - 3.33329491539e-1) * z * xp + xp
    res = y0 + r
    return jnp.where(x < 0.0, -res, res)


def _i32const(u):
    u &= 0xFFFFFFFF
    return jnp.int32(u - (1 << 32) if u >= (1 << 31) else u)


def _body(st_ref, pbt_ref, gb_ref, glr_ref, th_ref, out_ref, *, N, C, M, K):
    b = pl.program_id(0)
    s = st_ref[0]
    pbt = pbt_ref[0]
    gb = gb_ref[0]
    glr = glr_ref[0]
    th = th_ref[0, 0]

    mx = jnp.max(s, axis=0, keepdims=True)
    lse = mx + jnp.log(jnp.sum(jnp.exp(s - mx), axis=0, keepdims=True))
    e_abs = jnp.exp(-jnp.abs(s))
    sig_all = jnp.where(s >= 0.0, 1.0, e_abs) / (1.0 + e_abs)
    sp_all = jnp.maximum(s, 0.0) + jnp.log1p(e_abs)
    s0 = jnp.sum(0.2 * sig_all * sp_all)

    cio = jax.lax.broadcasted_iota(jnp.int32, (C, M), 0).astype(jnp.float32)
    onehot_c = (cio == glr).astype(jnp.float32)
    L = jax.lax.dot_general(onehot_c, s, (((0,), (0,)), ((), ())),
                            preferred_element_type=jnp.float32)

    counts = jnp.sum(onehot_c, axis=1, keepdims=True)
    cwc = 1.0 / (counts + 1e-6)
    cwc = cwc / jnp.max(cwc)
    cwg = jnp.sum(onehot_c * cwc, axis=0, keepdims=True)

    px0 = pbt[0:1, :]
    py0 = pbt[1:2, :]
    px1 = pbt[2:3, :]
    py1 = pbt[3:4, :]
    gx0 = gb[:, 0:1]
    gy0 = gb[:, 1:2]
    gx1 = gb[:, 2:3]
    gy1 = gb[:, 3:4]
    area_p = (px1 - px0) * (py1 - py0)
    area_g = (gx1 - gx0) * (gy1 - gy0)
    iw = jnp.maximum(jnp.minimum(px1, gx1) - jnp.maximum(px0, gx0), 0.0)
    ih = jnp.maximum(jnp.minimum(py1, gy1) - jnp.maximum(py0, gy0), 0.0)
    inter = iw * ih
    iou = inter / (area_p + area_g - inter + _EPS)
    cost = jnp.where(iou > th, lse - L, 1e5) - 3.0 * iou

    ki = jax.lax.bitcast_convert_type(cost, jnp.int32)
    skey = ki ^ ((ki >> 31) & jnp.int32(0x7FFFFFFF))
    imin = _i32const(0x80000000)
    ukey = skey ^ imin

    ones_n = jnp.ones((N, 1), jnp.float32)
    dn = (((1,), (0,)), ((), ()))
    prefix = jnp.zeros((M, 1), jnp.int32)
    kleft = jnp.full((M, 1), K, jnp.float32)
    active = jnp.ones((M, N), jnp.bool_)
    for bit in range(31, -1, -1):
        bt = _i32const(1 << bit)
        iszero = (ukey & bt) == 0
        m0f = (active & iszero).astype(jnp.float32)
        cnt0 = jax.lax.dot_general(m0f, ones_n, dn,
                                   preferred_element_type=jnp.float32)
        take1 = kleft > cnt0
        prefix = jnp.where(take1, prefix | bt, prefix)
        kleft = jnp.where(take1, kleft - cnt0, kleft)
        active = active & (iszero ^ take1)
    t_s = prefix ^ imin
    r = kleft

    eqt = active
    eqtf = active.astype(jnp.float32)
    rows = jax.lax.broadcasted_iota(jnp.int32, (M, N), 1)
    lo = jnp.zeros((M, 1), jnp.int32)
    hi = jnp.full((M, 1), N, jnp.int32)
    for _ in range(15):
        mid = (lo + hi) // 2
        c = jax.lax.dot_general(eqtf * (rows < mid).astype(jnp.float32),
                                ones_n, dn, preferred_element_type=jnp.float32)
        ge = c >= r
        hi = jnp.where(ge, mid, hi)
        lo = jnp.where(ge, lo, mid)
    member = (skey < t_s) | (eqt & (rows < hi))

    mf = member.astype(jnp.float32)
    pos = jnp.sum(mf, axis=0, keepdims=True) > 0.0
    m_iota = jax.lax.broadcasted_iota(jnp.int32, (M, N), 0)
    idxm = jnp.min(jnp.where(member, m_iota, jnp.int32(1 << 30)), axis=0, keepdims=True)
    mg = jnp.where(pos, idxm, 0)
    onehot_m = (m_iota == mg).astype(jnp.float32)

    mgt = jax.lax.dot_general(gb, onehot_m, (((0,), (0,)), ((), ())),
                              preferred_element_type=jnp.float32)
    cw_n = jax.lax.dot_general(cwg, onehot_m, (((1,), (0,)), ((), ())),
                               preferred_element_type=jnp.float32)
    l_sel = jnp.sum(L * onehot_m, axis=0, keepdims=True)

    mx0 = mgt[0:1, :]
    my0 = mgt[1:2, :]
    mx1 = mgt[2:3, :]
    my1 = mgt[3:4, :]

    iw2 = jnp.maximum(jnp.minimum(px1, mx1) - jnp.maximum(px0, mx0), 0.0)
    ih2 = jnp.maximum(jnp.minimum(py1, my1) - jnp.maximum(py0, my0), 0.0)
    inter2 = iw2 * ih2
    ap = (px1 - px0) * (py1 - py0)
    ag = (mx1 - mx0) * (my1 - my0)
    union = ap + ag - inter2 + _EPS
    iou2 = inter2 / union
    cw_e = jnp.maximum(px1, mx1) - jnp.minimum(px0, mx0)
    ch_e = jnp.maximum(py1, my1) - jnp.minimum(py0, my0)
    c2 = cw_e * cw_e + ch_e * ch_e + _EPS
    rho2 = ((px0 + px1 - mx0 - mx1) ** 2 + (py0 + py1 - my0 - my1) ** 2) / 4.0
    wp = px1 - px0 + _EPS
    hp = py1 - py0 + _EPS
    wg = mx1 - mx0 + _EPS
    hg = my1 - my0 + _EPS
    v = (4.0 / (_PI * _PI)) * (_atan_pos(wg / hg) - _atan_pos(wp / hp)) ** 2
    alpha = v / (v - iou2 + 1.0 + _EPS)
    ciou = iou2 - rho2 / c2 - alpha * v

    fmask = pos.astype(jnp.float32)
    cnt = jnp.sum(fmask)
    ciou_t = jnp.clip(ciou, 0.0, 1.0)
    tv = jnp.where(pos, 0.7 + 0.25 * ciou_t, 0.0)

    p_l = 1.0 / (1.0 + jnp.exp(-l_sel))
    sp_l = jnp.maximum(l_sel, 0.0) + jnp.log1p(jnp.exp(-jnp.abs(l_sel)))
    ce = sp_l - l_sel * tv
    p_t = p_l * tv + (1.0 - p_l) * (1.0 - tv)
    a_t = 0.8 * tv + 0.2 * (1.0 - tv)
    term = a_t * (1.0 - p_t) * ce
    term0 = 0.2 * p_l * sp_l
    corr = jnp.sum(term - term0)

    lc_b = (s0 + corr) / (N * C) * (jnp.sum(fmask * cw_n) / cnt)

    dcx = (px0 + px1) * 0.5 - (mx0 + mx1) * 0.5
    dcy = (py0 + py1) * 0.5 - (my0 + my1) * 0.5
    dwx = ((px1 - px0) - (mx1 - mx0)) * 0.5
    dwy = ((py1 - py0) - (my1 - my0)) * 0.5
    d2 = dcx * dcx + dcy * dcy + dwx * dwx + dwy * dwy
    gwd = jnp.log1p(jnp.sqrt(d2 + _EPS))
    area_m = (mx1 - mx0) * (my1 - my0) + 1e-6
    sw = jnp.clip(1.0 / area_m, 0.1, 10.0)
    lb_b = jnp.sum(fmask * gwd * sw) / cnt

    iwgt = jnp.maximum(ciou * ciou, 0.01)
    li_b = jnp.sum(fmask * (1.0 - ciou) * iwgt) / cnt

    lane = jax.lax.broadcasted_iota(jnp.int32, (1, 128), 1)
    vec = (jnp.where(lane == 0, lc_b, 0.0)
           + jnp.where(lane == 1, lb_b, 0.0)
           + jnp.where(lane == 2, li_b, 0.0))

    @pl.when(b == 0)
    def _init():
        out_ref[...] = jnp.zeros_like(out_ref)

    out_ref[...] += vec


def kernel(pred_scores, pred_boxes, target_boxes, target_labels, epoch, epochs, warmup_epoch):
    B, N, C = pred_scores.shape
    M = target_boxes.shape[1]
    K = max(1, min(int(N * 0.2), M * 10))
    K = min(K, N)

    s_t = jnp.transpose(pred_scores, (0, 2, 1))
    pbt = jnp.transpose(pred_boxes, (0, 2, 1))
    glr = target_labels.astype(jnp.float32).reshape(B, 1, M)
    th = jnp.where(jnp.asarray(epoch) < jnp.asarray(warmup_epoch),
                   jnp.float32(0.25 / 10000.0), jnp.float32(0.25))
    th_arr = jnp.broadcast_to(th.astype(jnp.float32), (1, 128))

    out = pl.pallas_call(
        functools.partial(_body, N=N, C=C, M=M, K=K),
        grid=(B,),
        in_specs=[
            pl.BlockSpec((1, C, N), lambda b: (b, 0, 0)),
            pl.BlockSpec((1, 4, N), lambda b: (b, 0, 0)),
            pl.BlockSpec((1, M, 4), lambda b: (b, 0, 0)),
            pl.BlockSpec((1, 1, M), lambda b: (b, 0, 0)),
            pl.BlockSpec((1, 128), lambda b: (0, 0)),
        ],
        out_specs=pl.BlockSpec((1, 128), lambda b: (0, 0)),
        out_shape=jax.ShapeDtypeStruct((1, 128), jnp.float32),
    )(s_t, pbt, target_boxes, glr, th_arr)

    lc = out[0, 0] / B
    lb = out[0, 1] / B
    li = out[0, 2] / B
    total = lc + lb + li
    return (total, lc, lb, li)


# speedup vs baseline: 1.0781x; 1.0781x over previous
import functools

import jax
import jax.numpy as jnp
import numpy as np
from jax.experimental import pallas as pl

_EPS = 1e-7
_PI = float(np.pi)


def _atan_pos(x):
    t = jnp.abs(x)
    sel1 = t > 2.414213562373095
    sel2 = t > 0.4142135623730950
    xp = jnp.where(sel1, -1.0 / (t + 1e-30), jnp.where(sel2, (t - 1.0) / (t + 1.0), t))
    y0 = jnp.where(sel1, _PI / 2.0, jnp.where(sel2, _PI / 4.0, 0.0))
    z = xp * xp
    r = ((8.05374449538e-2 * z - 1.38776856032e-1) * z + 1.99777106478e-1) * z
    r = (r - 3.33329491539e-1) * z * xp + xp
    res = y0 + r
    return jnp.where(x < 0.0, -res, res)


def _i32const(u):
    u &= 0xFFFFFFFF
    return jnp.int32(u - (1 << 32) if u >= (1 << 31) else u)


def _body(st_ref, pbt_ref, gb_ref, glr_ref, th_ref, out_ref, *, N, C, M, K):
    b = pl.program_id(0)
    s = st_ref[0]
    pbt = pbt_ref[0]
    gb = gb_ref[0]
    glr = glr_ref[0]
    th = th_ref[0, 0]

    mx = jnp.max(s, axis=0, keepdims=True)
    lse = mx + jnp.log(jnp.sum(jnp.exp(s - mx), axis=0, keepdims=True))
    e_abs = jnp.exp(-jnp.abs(s))
    sig_all = jnp.where(s >= 0.0, 1.0, e_abs) / (1.0 + e_abs)
    sp_all = jnp.maximum(s, 0.0) + jnp.log1p(e_abs)
    s0 = jnp.sum(0.2 * sig_all * sp_all)

    cio = jax.lax.broadcasted_iota(jnp.int32, (C, M), 0).astype(jnp.float32)
    onehot_c = (cio == glr).astype(jnp.float32)
    L = jax.lax.dot_general(onehot_c, s, (((0,), (0,)), ((), ())),
                            preferred_element_type=jnp.float32)

    counts = jnp.sum(onehot_c, axis=1, keepdims=True)
    cwc = 1.0 / (counts + 1e-6)
    cwc = cwc / jnp.max(cwc)
    cwg = jnp.sum(onehot_c * cwc, axis=0, keepdims=True)

    px0 = pbt[0:1, :]
    py0 = pbt[1:2, :]
    px1 = pbt[2:3, :]
    py1 = pbt[3:4, :]
    gx0 = gb[:, 0:1]
    gy0 = gb[:, 1:2]
    gx1 = gb[:, 2:3]
    gy1 = gb[:, 3:4]
    area_p = (px1 - px0) * (py1 - py0)
    area_g = (gx1 - gx0) * (gy1 - gy0)
    iw = jnp.maximum(jnp.minimum(px1, gx1) - jnp.maximum(px0, gx0), 0.0)
    ih = jnp.maximum(jnp.minimum(py1, gy1) - jnp.maximum(py0, gy0), 0.0)
    inter = iw * ih
    iou = inter / (area_p + area_g - inter + _EPS)
    cost = jnp.where(iou > th, lse - L, 1e5) - 3.0 * iou

    ki = jax.lax.bitcast_convert_type(cost, jnp.int32)
    skey = ki ^ ((ki >> 31) & jnp.int32(0x7FFFFFFF))
    imin = _i32const(0x80000000)
    ukey = skey ^ imin

    prefix = jnp.zeros((M, 1), jnp.int32)
    kleft = jnp.full((M, 1), K, jnp.int32)
    active = jnp.ones((M, N), jnp.bool_)
    for bit in range(31, -1, -1):
        bt = _i32const(1 << bit)
        iszero = (ukey & bt) == 0
        cnt0 = jnp.sum((active & iszero).astype(jnp.int32), axis=1, keepdims=True)
        take1 = kleft > cnt0
        prefix = jnp.where(take1, prefix | bt, prefix)
        kleft = jnp.where(take1, kleft - cnt0, kleft)
        active = active & (iszero ^ take1)
    t_s = prefix ^ imin
    r = kleft

    eqt = active
    rows = jax.lax.broadcasted_iota(jnp.int32, (M, N), 1)
    lo = jnp.zeros((M, 1), jnp.int32)
    hi = jnp.full((M, 1), N, jnp.int32)
    for _ in range(15):
        mid = (lo + hi) // 2
        c = jnp.sum((eqt & (rows < mid)).astype(jnp.int32), axis=1, keepdims=True)
        ge = c >= r
        hi = jnp.where(ge, mid, hi)
        lo = jnp.where(ge, lo, mid)
    member = (skey < t_s) | (eqt & (rows < hi))

    mf = member.astype(jnp.float32)
    pos = jnp.sum(mf, axis=0, keepdims=True) > 0.0
    m_iota = jax.lax.broadcasted_iota(jnp.int32, (M, N), 0)
    idxm = jnp.min(jnp.where(member, m_iota, jnp.int32(1 << 30)), axis=0, keepdims=True)
    mg = jnp.where(pos, idxm, 0)
    onehot_m = (m_iota == mg).astype(jnp.float32)

    mgt = jax.lax.dot_general(gb, onehot_m, (((0,), (0,)), ((), ())),
                              preferred_element_type=jnp.float32)
    cw_n = jax.lax.dot_general(cwg, onehot_m, (((1,), (0,)), ((), ())),
                               preferred_element_type=jnp.float32)
    l_sel = jnp.sum(L * onehot_m, axis=0, keepdims=True)

    mx0 = mgt[0:1, :]
    my0 = mgt[1:2, :]
    mx1 = mgt[2:3, :]
    my1 = mgt[3:4, :]

    iw2 = jnp.maximum(jnp.minimum(px1, mx1) - jnp.maximum(px0, mx0), 0.0)
    ih2 = jnp.maximum(jnp.minimum(py1, my1) - jnp.maximum(py0, my0), 0.0)
    inter2 = iw2 * ih2
    ap = (px1 - px0) * (py1 - py0)
    ag = (mx1 - mx0) * (my1 - my0)
    union = ap + ag - inter2 + _EPS
    iou2 = inter2 / union
    cw_e = jnp.maximum(px1, mx1) - jnp.minimum(px0, mx0)
    ch_e = jnp.maximum(py1, my1) - jnp.minimum(py0, my0)
    c2 = cw_e * cw_e + ch_e * ch_e + _EPS
    rho2 = ((px0 + px1 - mx0 - mx1) ** 2 + (py0 + py1 - my0 - my1) ** 2) / 4.0
    wp = px1 - px0 + _EPS
    hp = py1 - py0 + _EPS
    wg = mx1 - mx0 + _EPS
    hg = my1 - my0 + _EPS
    v = (4.0 / (_PI * _PI)) * (_atan_pos(wg / hg) - _atan_pos(wp / hp)) ** 2
    alpha = v / (v - iou2 + 1.0 + _EPS)
    ciou = iou2 - rho2 / c2 - alpha * v

    fmask = pos.astype(jnp.float32)
    cnt = jnp.sum(fmask)
    ciou_t = jnp.clip(ciou, 0.0, 1.0)
    tv = jnp.where(pos, 0.7 + 0.25 * ciou_t, 0.0)

    p_l = 1.0 / (1.0 + jnp.exp(-l_sel))
    sp_l = jnp.maximum(l_sel, 0.0) + jnp.log1p(jnp.exp(-jnp.abs(l_sel)))
    ce = sp_l - l_sel * tv
    p_t = p_l * tv + (1.0 - p_l) * (1.0 - tv)
    a_t = 0.8 * tv + 0.2 * (1.0 - tv)
    term = a_t * (1.0 - p_t) * ce
    term0 = 0.2 * p_l * sp_l
    corr = jnp.sum(term - term0)

    lc_b = (s0 + corr) / (N * C) * (jnp.sum(fmask * cw_n) / cnt)

    dcx = (px0 + px1) * 0.5 - (mx0 + mx1) * 0.5
    dcy = (py0 + py1) * 0.5 - (my0 + my1) * 0.5
    dwx = ((px1 - px0) - (mx1 - mx0)) * 0.5
    dwy = ((py1 - py0) - (my1 - my0)) * 0.5
    d2 = dcx * dcx + dcy * dcy + dwx * dwx + dwy * dwy
    gwd = jnp.log1p(jnp.sqrt(d2 + _EPS))
    area_m = (mx1 - mx0) * (my1 - my0) + 1e-6
    sw = jnp.clip(1.0 / area_m, 0.1, 10.0)
    lb_b = jnp.sum(fmask * gwd * sw) / cnt

    iwgt = jnp.maximum(ciou * ciou, 0.01)
    li_b = jnp.sum(fmask * (1.0 - ciou) * iwgt) / cnt

    lane = jax.lax.broadcasted_iota(jnp.int32, (1, 128), 1)
    vec = (jnp.where(lane == 0, lc_b, 0.0)
           + jnp.where(lane == 1, lb_b, 0.0)
           + jnp.where(lane == 2, li_b, 0.0))

    @pl.when(b == 0)
    def _init():
        out_ref[...] = jnp.zeros_like(out_ref)

    out_ref[...] += vec


def kernel(pred_scores, pred_boxes, target_boxes, target_labels, epoch, epochs, warmup_epoch):
    B, N, C = pred_scores.shape
    M = target_boxes.shape[1]
    K = max(1, min(int(N * 0.2), M * 10))
    K = min(K, N)

    s_t = jnp.transpose(pred_scores, (0, 2, 1))
    pbt = jnp.transpose(pred_boxes, (0, 2, 1))
    glr = target_labels.astype(jnp.float32).reshape(B, 1, M)
    th = jnp.where(jnp.asarray(epoch) < jnp.asarray(warmup_epoch),
                   jnp.float32(0.25 / 10000.0), jnp.float32(0.25))
    th_arr = jnp.broadcast_to(th.astype(jnp.float32), (1, 128))

    out = pl.pallas_call(
        functools.partial(_body, N=N, C=C, M=M, K=K),
        grid=(B,),
        in_specs=[
            pl.BlockSpec((1, C, N), lambda b: (b, 0, 0)),
            pl.BlockSpec((1, 4, N), lambda b: (b, 0, 0)),
            pl.BlockSpec((1, M, 4), lambda b: (b, 0, 0)),
            pl.BlockSpec((1, 1, M), lambda b: (b, 0, 0)),
            pl.BlockSpec((1, 128), lambda b: (0, 0)),
        ],
        out_specs=pl.BlockSpec((1, 128), lambda b: (0, 0)),
        out_shape=jax.ShapeDtypeStruct((1, 128), jnp.float32),
    )(s_t, pbt, target_boxes, glr, th_arr)

    lc = out[0, 0] / B
    lb = out[0, 1] / B
    li = out[0, 2] / B
    total = lc + lb + li
    return (total, lc, lb, li)


# speedup vs baseline: 1.3144x; 1.2192x over previous
import functools

import jax
import jax.numpy as jnp
import numpy as np
from jax.experimental import pallas as pl

_EPS = 1e-7
_PI = float(np.pi)


def _atan_pos(x):
    t = jnp.abs(x)
    sel1 = t > 2.414213562373095
    sel2 = t > 0.4142135623730950
    xp = jnp.where(sel1, -1.0 / (t + 1e-30), jnp.where(sel2, (t - 1.0) / (t + 1.0), t))
    y0 = jnp.where(sel1, _PI / 2.0, jnp.where(sel2, _PI / 4.0, 0.0))
    z = xp * xp
    r = ((8.05374449538e-2 * z - 1.38776856032e-1) * z + 1.99777106478e-1) * z
    r = (r - 3.33329491539e-1) * z * xp + xp
    res = y0 + r
    return jnp.where(x < 0.0, -res, res)


def _i32const(u):
    u &= 0xFFFFFFFF
    return jnp.int32(u - (1 << 32) if u >= (1 << 31) else u)


def _body(st_ref, pbt_ref, gb_ref, glr_ref, th_ref, out_ref, *, N, C, M, K):
    b = pl.program_id(0)
    s = st_ref[0]
    pbt = pbt_ref[0]
    gb = gb_ref[0]
    glr = glr_ref[0]
    th = th_ref[0, 0]

    mx = jnp.max(s, axis=0, keepdims=True)
    lse = mx + jnp.log(jnp.sum(jnp.exp(s - mx), axis=0, keepdims=True))
    e_abs = jnp.exp(-jnp.abs(s))
    sig_all = jnp.where(s >= 0.0, 1.0, e_abs) / (1.0 + e_abs)
    sp_all = jnp.maximum(s, 0.0) + jnp.log1p(e_abs)
    s0 = jnp.sum(0.2 * sig_all * sp_all)

    cio = jax.lax.broadcasted_iota(jnp.int32, (C, M), 0).astype(jnp.float32)
    onehot_c = (cio == glr).astype(jnp.float32)
    L = jax.lax.dot_general(onehot_c, s, (((0,), (0,)), ((), ())),
                            preferred_element_type=jnp.float32)

    counts = jnp.sum(onehot_c, axis=1, keepdims=True)
    cwc = 1.0 / (counts + 1e-6)
    cwc = cwc / jnp.max(cwc)
    cwg = jnp.sum(onehot_c * cwc, axis=0, keepdims=True)

    px0 = pbt[0:1, :]
    py0 = pbt[1:2, :]
    px1 = pbt[2:3, :]
    py1 = pbt[3:4, :]
    gx0 = gb[:, 0:1]
    gy0 = gb[:, 1:2]
    gx1 = gb[:, 2:3]
    gy1 = gb[:, 3:4]
    area_p = (px1 - px0) * (py1 - py0)
    area_g = (gx1 - gx0) * (gy1 - gy0)
    iw = jnp.maximum(jnp.minimum(px1, gx1) - jnp.maximum(px0, gx0), 0.0)
    ih = jnp.maximum(jnp.minimum(py1, gy1) - jnp.maximum(py0, gy0), 0.0)
    inter = iw * ih
    iou = inter / (area_p + area_g - inter + _EPS)
    cost = jnp.where(iou > th, lse - L, 1e5) - 3.0 * iou

    ki = jax.lax.bitcast_convert_type(cost, jnp.int32)
    skey = ki ^ ((ki >> 31) & jnp.int32(0x7FFFFFFF))
    imin = _i32const(0x80000000)
    ukey = skey ^ imin

    prefix = jnp.zeros((M, 1), jnp.int32)
    kleft = jnp.full((M, 1), K, jnp.int32)
    for bit in range(31, -1, -1):
        hm = _i32const((0xFFFFFFFF << (bit + 1)))
        bt = _i32const(1 << bit)
        match0 = ((ukey & hm) == prefix) & ((ukey & bt) == 0)
        cnt0 = jnp.sum(match0.astype(jnp.int32), axis=1, keepdims=True)
        take1 = kleft > cnt0
        prefix = jnp.where(take1, prefix | bt, prefix)
        kleft = jnp.where(take1, kleft - cnt0, kleft)
    t_s = prefix ^ imin
    r = kleft

    eqt = skey == t_s
    rows = jax.lax.broadcasted_iota(jnp.int32, (M, N), 1)
    lo = jnp.zeros((M, 1), jnp.int32)
    hi = jnp.full((M, 1), N, jnp.int32)
    for _ in range(15):
        mid = (lo + hi) // 2
        c = jnp.sum((eqt & (rows < mid)).astype(jnp.int32), axis=1, keepdims=True)
        ge = c >= r
        hi = jnp.where(ge, mid, hi)
        lo = jnp.where(ge, lo, mid)
    member = (skey < t_s) | (eqt & (rows < hi))

    mf = member.astype(jnp.float32)
    pos = jnp.sum(mf, axis=0, keepdims=True) > 0.0
    m_iota = jax.lax.broadcasted_iota(jnp.int32, (M, N), 0)
    idxm = jnp.min(jnp.where(member, m_iota, jnp.int32(1 << 30)), axis=0, keepdims=True)
    mg = jnp.where(pos, idxm, 0)
    onehot_m = (m_iota == mg).astype(jnp.float32)

    mgt = jax.lax.dot_general(gb, onehot_m, (((0,), (0,)), ((), ())),
                              preferred_element_type=jnp.float32)
    cw_n = jax.lax.dot_general(cwg, onehot_m, (((1,), (0,)), ((), ())),
                               preferred_element_type=jnp.float32)
    l_sel = jnp.sum(L * onehot_m, axis=0, keepdims=True)

    mx0 = mgt[0:1, :]
    my0 = mgt[1:2, :]
    mx1 = mgt[2:3, :]
    my1 = mgt[3:4, :]

    iw2 = jnp.maximum(jnp.minimum(px1, mx1) - jnp.maximum(px0, mx0), 0.0)
    ih2 = jnp.maximum(jnp.minimum(py1, my1) - jnp.maximum(py0, my0), 0.0)
    inter2 = iw2 * ih2
    ap = (px1 - px0) * (py1 - py0)
    ag = (mx1 - mx0) * (my1 - my0)
    union = ap + ag - inter2 + _EPS
    iou2 = inter2 / union
    cw_e = jnp.maximum(px1, mx1) - jnp.minimum(px0, mx0)
    ch_e = jnp.maximum(py1, my1) - jnp.minimum(py0, my0)
    c2 = cw_e * cw_e + ch_e * ch_e + _EPS
    rho2 = ((px0 + px1 - mx0 - mx1) ** 2 + (py0 + py1 - my0 - my1) ** 2) / 4.0
    wp = px1 - px0 + _EPS
    hp = py1 - py0 + _EPS
    wg = mx1 - mx0 + _EPS
    hg = my1 - my0 + _EPS
    v = (4.0 / (_PI * _PI)) * (_atan_pos(wg / hg) - _atan_pos(wp / hp)) ** 2
    alpha = v / (v - iou2 + 1.0 + _EPS)
    ciou = iou2 - rho2 / c2 - alpha * v

    fmask = pos.astype(jnp.float32)
    cnt = jnp.sum(fmask)
    ciou_t = jnp.clip(ciou, 0.0, 1.0)
    tv = jnp.where(pos, 0.7 + 0.25 * ciou_t, 0.0)

    p_l = 1.0 / (1.0 + jnp.exp(-l_sel))
    sp_l = jnp.maximum(l_sel, 0.0) + jnp.log1p(jnp.exp(-jnp.abs(l_sel)))
    ce = sp_l - l_sel * tv
    p_t = p_l * tv + (1.0 - p_l) * (1.0 - tv)
    a_t = 0.8 * tv + 0.2 * (1.0 - tv)
    term = a_t * (1.0 - p_t) * ce
    term0 = 0.2 * p_l * sp_l
    corr = jnp.sum(term - term0)

    lc_b = (s0 + corr) / (N * C) * (jnp.sum(fmask * cw_n) / cnt)

    dcx = (px0 + px1) * 0.5 - (mx0 + mx1) * 0.5
    dcy = (py0 + py1) * 0.5 - (my0 + my1) * 0.5
    dwx = ((px1 - px0) - (mx1 - mx0)) * 0.5
    dwy = ((py1 - py0) - (my1 - my0)) * 0.5
    d2 = dcx * dcx + dcy * dcy + dwx * dwx + dwy * dwy
    gwd = jnp.log1p(jnp.sqrt(d2 + _EPS))
    area_m = (mx1 - mx0) * (my1 - my0) + 1e-6
    sw = jnp.clip(1.0 / area_m, 0.1, 10.0)
    lb_b = jnp.sum(fmask * gwd * sw) / cnt

    iwgt = jnp.maximum(ciou * ciou, 0.01)
    li_b = jnp.sum(fmask * (1.0 - ciou) * iwgt) / cnt

    lane = jax.lax.broadcasted_iota(jnp.int32, (1, 128), 1)
    vec = (jnp.where(lane == 0, lc_b, 0.0)
           + jnp.where(lane == 1, lb_b, 0.0)
           + jnp.where(lane == 2, li_b, 0.0))

    @pl.when(b == 0)
    def _init():
        out_ref[...] = jnp.zeros_like(out_ref)

    out_ref[...] += vec


def kernel(pred_scores, pred_boxes, target_boxes, target_labels, epoch, epochs, warmup_epoch):
    B, N, C = pred_scores.shape
    M = target_boxes.shape[1]
    K = max(1, min(int(N * 0.2), M * 10))
    K = min(K, N)

    s_t = jnp.transpose(pred_scores, (0, 2, 1))
    pbt = jnp.transpose(pred_boxes, (0, 2, 1))
    glr = target_labels.astype(jnp.float32).reshape(B, 1, M)
    th = jnp.where(jnp.asarray(epoch) < jnp.asarray(warmup_epoch),
                   jnp.float32(0.25 / 10000.0), jnp.float32(0.25))
    th_arr = jnp.broadcast_to(th.astype(jnp.float32), (1, 128))

    out = pl.pallas_call(
        functools.partial(_body, N=N, C=C, M=M, K=K),
        grid=(B,),
        in_specs=[
            pl.BlockSpec((1, C, N), lambda b: (b, 0, 0)),
            pl.BlockSpec((1, 4, N), lambda b: (b, 0, 0)),
            pl.BlockSpec((1, M, 4), lambda b: (b, 0, 0)),
            pl.BlockSpec((1, 1, M), lambda b: (b, 0, 0)),
            pl.BlockSpec((1, 128), lambda b: (0, 0)),
        ],
        out_specs=pl.BlockSpec((1, 128), lambda b: (0, 0)),
        out_shape=jax.ShapeDtypeStruct((1, 128), jnp.float32),
    )(s_t, pbt, target_boxes, glr, th_arr)

    lc = out[0, 0] / B
    lb = out[0, 1] / B
    li = out[0, 2] / B
    total = lc + lb + li
    return (total, lc, lb, li)


# speedup vs baseline: 1.3335x; 1.0145x over previous
import functools

import jax
import jax.numpy as jnp
import numpy as np
from jax.experimental import pallas as pl

_EPS = 1e-7
_PI = float(np.pi)


def _atan_pos(x):
    t = jnp.abs(x)
    sel1 = t > 2.414213562373095
    sel2 = t > 0.4142135623730950
    xp = jnp.where(sel1, -1.0 / (t + 1e-30), jnp.where(sel2, (t - 1.0) / (t + 1.0), t))
    y0 = jnp.where(sel1, _PI / 2.0, jnp.where(sel2, _PI / 4.0, 0.0))
    z = xp * xp
    r = ((8.05374449538e-2 * z - 1.38776856032e-1) * z + 1.99777106478e-1) * z
    r = (r - 3.33329491539e-1) * z * xp + xp
    res = y0 + r
    return jnp.where(x < 0.0, -res, res)


def _i32const(u):
    u &= 0xFFFFFFFF
    return jnp.int32(u - (1 << 32) if u >= (1 << 31) else u)


def _body(st_ref, pbt_ref, gb_ref, glr_ref, th_ref, out_ref, *, N, C, M, K):
    b = pl.program_id(0)
    s = st_ref[0]
    pbt = pbt_ref[0]
    gb = gb_ref[0]
    glr = glr_ref[0]
    th = th_ref[0, 0]

    mx = jnp.max(s, axis=0, keepdims=True)
    lse = mx + jnp.log(jnp.sum(jnp.exp(s - mx), axis=0, keepdims=True))
    sig_all = 1.0 / (1.0 + jnp.exp(-s))
    sp_all = jnp.maximum(s, 0.0) + jnp.log1p(jnp.exp(-jnp.abs(s)))
    s0 = jnp.sum(0.2 * sig_all * sp_all)

    cio = jax.lax.broadcasted_iota(jnp.int32, (C, M), 0).astype(jnp.float32)
    onehot_c = (cio == glr).astype(jnp.float32)
    L = jax.lax.dot_general(onehot_c, s, (((0,), (0,)), ((), ())),
                            preferred_element_type=jnp.float32)

    counts = jnp.sum(onehot_c, axis=1, keepdims=True)
    cwc = 1.0 / (counts + 1e-6)
    cwc = cwc / jnp.max(cwc)
    cwg = jnp.sum(onehot_c * cwc, axis=0, keepdims=True)

    px0 = pbt[0:1, :]
    py0 = pbt[1:2, :]
    px1 = pbt[2:3, :]
    py1 = pbt[3:4, :]
    gx0 = gb[:, 0:1]
    gy0 = gb[:, 1:2]
    gx1 = gb[:, 2:3]
    gy1 = gb[:, 3:4]
    area_p = (px1 - px0) * (py1 - py0)
    area_g = (gx1 - gx0) * (gy1 - gy0)
    iw = jnp.maximum(jnp.minimum(px1, gx1) - jnp.maximum(px0, gx0), 0.0)
    ih = jnp.maximum(jnp.minimum(py1, gy1) - jnp.maximum(py0, gy0), 0.0)
    inter = iw * ih
    iou = inter / (area_p + area_g - inter + _EPS)
    cost = jnp.where(iou > th, lse - L, 1e5) - 3.0 * iou

    ki = jax.lax.bitcast_convert_type(cost, jnp.int32)
    skey = ki ^ ((ki >> 31) & jnp.int32(0x7FFFFFFF))
    imin = _i32const(0x80000000)
    ukey = skey ^ imin

    prefix = jnp.zeros((M, 1), jnp.int32)
    kleft = jnp.full((M, 1), K, jnp.int32)
    for bit in range(31, -1, -1):
        hm = _i32const((0xFFFFFFFF << (bit + 1)))
        bt = _i32const(1 << bit)
        match0 = ((ukey & hm) == prefix) & ((ukey & bt) == 0)
        cnt0 = jnp.sum(match0.astype(jnp.int32), axis=1, keepdims=True)
        take1 = kleft > cnt0
        prefix = jnp.where(take1, prefix | bt, prefix)
        kleft = jnp.where(take1, kleft - cnt0, kleft)
    t_s = prefix ^ imin
    r = kleft

    eqt = skey == t_s
    rows = jax.lax.broadcasted_iota(jnp.int32, (M, N), 1)
    lo = jnp.zeros((M, 1), jnp.int32)
    hi = jnp.full((M, 1), N, jnp.int32)
    for _ in range(15):
        mid = (lo + hi) // 2
        c = jnp.sum((eqt & (rows < mid)).astype(jnp.int32), axis=1, keepdims=True)
        ge = c >= r
        hi = jnp.where(ge, mid, hi)
        lo = jnp.where(ge, lo, mid)
    member = (skey < t_s) | (eqt & (rows < hi))

    mf = member.astype(jnp.float32)
    pos = jnp.sum(mf, axis=0, keepdims=True) > 0.0
    m_iota = jax.lax.broadcasted_iota(jnp.int32, (M, N), 0)
    idxm = jnp.min(jnp.where(member, m_iota, jnp.int32(1 << 30)), axis=0, keepdims=True)
    mg = jnp.where(pos, idxm, 0)
    onehot_m = (m_iota == mg).astype(jnp.float32)

    mgt = jax.lax.dot_general(gb, onehot_m, (((0,), (0,)), ((), ())),
                              preferred_element_type=jnp.float32)
    cw_n = jax.lax.dot_general(cwg, onehot_m, (((1,), (0,)), ((), ())),
                               preferred_element_type=jnp.float32)
    l_sel = jnp.sum(L * onehot_m, axis=0, keepdims=True)

    mx0 = mgt[0:1, :]
    my0 = mgt[1:2, :]
    mx1 = mgt[2:3, :]
    my1 = mgt[3:4, :]

    iw2 = jnp.maximum(jnp.minimum(px1, mx1) - jnp.maximum(px0, mx0), 0.0)
    ih2 = jnp.maximum(jnp.minimum(py1, my1) - jnp.maximum(py0, my0), 0.0)
    inter2 = iw2 * ih2
    ap = (px1 - px0) * (py1 - py0)
    ag = (mx1 - mx0) * (my1 - my0)
    union = ap + ag - inter2 + _EPS
    iou2 = inter2 / union
    cw_e = jnp.maximum(px1, mx1) - jnp.minimum(px0, mx0)
    ch_e = jnp.maximum(py1, my1) - jnp.minimum(py0, my0)
    c2 = cw_e * cw_e + ch_e * ch_e + _EPS
    rho2 = ((px0 + px1 - mx0 - mx1) ** 2 + (py0 + py1 - my0 - my1) ** 2) / 4.0
    wp = px1 - px0 + _EPS
    hp = py1 - py0 + _EPS
    wg = mx1 - mx0 + _EPS
    hg = my1 - my0 + _EPS
    v = (4.0 / (_PI * _PI)) * (_atan_pos(wg / hg) - _atan_pos(wp / hp)) ** 2
    alpha = v / (v - iou2 + 1.0 + _EPS)
    ciou = iou2 - rho2 / c2 - alpha * v

    fmask = pos.astype(jnp.float32)
    cnt = jnp.sum(fmask)
    ciou_t = jnp.clip(ciou, 0.0, 1.0)
    tv = jnp.where(pos, 0.7 + 0.25 * ciou_t, 0.0)

    p_l = 1.0 / (1.0 + jnp.exp(-l_sel))
    sp_l = jnp.maximum(l_sel, 0.0) + jnp.log1p(jnp.exp(-jnp.abs(l_sel)))
    ce = sp_l - l_sel * tv
    p_t = p_l * tv + (1.0 - p_l) * (1.0 - tv)
    a_t = 0.8 * tv + 0.2 * (1.0 - tv)
    term = a_t * (1.0 - p_t) * ce
    term0 = 0.2 * p_l * sp_l
    corr = jnp.sum(term - term0)

    lc_b = (s0 + corr) / (N * C) * (jnp.sum(fmask * cw_n) / cnt)

    dcx = (px0 + px1) * 0.5 - (mx0 + mx1) * 0.5
    dcy = (py0 + py1) * 0.5 - (my0 + my1) * 0.5
    dwx = ((px1 - px0) - (mx1 - mx0)) * 0.5
    dwy = ((py1 - py0) - (my1 - my0)) * 0.5
    d2 = dcx * dcx + dcy * dcy + dwx * dwx + dwy * dwy
    gwd = jnp.log1p(jnp.sqrt(d2 + _EPS))
    area_m = (mx1 - mx0) * (my1 - my0) + 1e-6
    sw = jnp.clip(1.0 / area_m, 0.1, 10.0)
    lb_b = jnp.sum(fmask * gwd * sw) / cnt

    iwgt = jnp.maximum(ciou * ciou, 0.01)
    li_b = jnp.sum(fmask * (1.0 - ciou) * iwgt) / cnt

    lane = jax.lax.broadcasted_iota(jnp.int32, (1, 128), 1)
    vec = (jnp.where(lane == 0, lc_b, 0.0)
           + jnp.where(lane == 1, lb_b, 0.0)
           + jnp.where(lane == 2, li_b, 0.0))

    @pl.when(b == 0)
    def _init():
        out_ref[...] = jnp.zeros_like(out_ref)

    out_ref[...] += vec


def kernel(pred_scores, pred_boxes, target_boxes, target_labels, epoch, epochs, warmup_epoch):
    B, N, C = pred_scores.shape
    M = target_boxes.shape[1]
    K = max(1, min(int(N * 0.2), M * 10))
    K = min(K, N)

    s_t = jnp.transpose(pred_scores, (0, 2, 1))
    pbt = jnp.transpose(pred_boxes, (0, 2, 1))
    glr = target_labels.astype(jnp.float32).reshape(B, 1, M)
    th = jnp.where(jnp.asarray(epoch) < jnp.asarray(warmup_epoch),
                   jnp.float32(0.25 / 10000.0), jnp.float32(0.25))
    th_arr = jnp.broadcast_to(th.astype(jnp.float32), (1, 128))

    out = pl.pallas_call(
        functools.partial(_body, N=N, C=C, M=M, K=K),
        grid=(B,),
        in_specs=[
            pl.BlockSpec((1, C, N), lambda b: (b, 0, 0)),
            pl.BlockSpec((1, 4, N), lambda b: (b, 0, 0)),
            pl.BlockSpec((1, M, 4), lambda b: (b, 0, 0)),
            pl.BlockSpec((1, 1, M), lambda b: (b, 0, 0)),
            pl.BlockSpec((1, 128), lambda b: (0, 0)),
        ],
        out_specs=pl.BlockSpec((1, 128), lambda b: (0, 0)),
        out_shape=jax.ShapeDtypeStruct((1, 128), jnp.float32),
    )(s_t, pbt, target_boxes, glr, th_arr)

    lc = out[0, 0] / B
    lb = out[0, 1] / B
    li = out[0, 2] / B
    total = lc + lb + li
    return (total, lc, lb, li)
